# baseline (speedup 1.0000x reference)
import functools

import jax
import jax.numpy as jnp
import numpy as np
from jax import lax
from jax.experimental import pallas as pl
from jax.experimental.pallas import tpu as pltpu
from jax.experimental.pallas import tpu_sc as plsc

N_NODES = 10000
D = 256
HALF = 128
N_GRAPHS = 5
N_EDGES = 160000
E_TOT = N_GRAPHS * N_EDGES
NSUB = 16
CHUNK = 32
N_CHUNKS = 1568
E_PAD = NSUB * N_CHUNKS * CHUNK
PK_ROWS = E_PAD // CHUNK
OUT_ROWS_PER_SUB = N_NODES // NSUB
IRING = 4

_PERM = np.arange(HALF)
_PERM = (_PERM // 32) * 32 + (_PERM % 32) // 2 + 16 * (_PERM % 2)


def _proj_body(x_ref, w_ref, o_ref):
    o_ref[...] = lax.dot_general(
        x_ref[...], w_ref[...], (((1,), (1,)), ((), ())),
        precision=lax.Precision.HIGHEST).astype(jnp.bfloat16)


def _project(x, W2):
    RB = 1000
    nblk = N_NODES // RB
    return pl.pallas_call(
        _proj_body,
        grid=(2, nblk),
        in_specs=[
            pl.BlockSpec((RB, D), lambda h, i: (i, 0)),
            pl.BlockSpec((HALF, D), lambda h, i: (h, 0)),
        ],
        out_specs=pl.BlockSpec((RB, HALF), lambda h, i: (h * nblk + i, 0)),
        out_shape=jax.ShapeDtypeStruct((2 * N_NODES, HALF), jnp.bfloat16),
    )(x, W2)


def _vals_body(v_ref, a_ref, o_ref):
    o_ref[...] = v_ref[...] * jax.nn.sigmoid(a_ref[...])


def _scale_vals(lap_val, alpha):
    return pl.pallas_call(
        _vals_body,
        in_specs=[
            pl.BlockSpec((N_GRAPHS, N_EDGES), lambda: (0, 0)),
            pl.BlockSpec((N_GRAPHS, 1), lambda: (0, 0)),
        ],
        out_specs=pl.BlockSpec((N_GRAPHS, N_EDGES), lambda: (0, 0)),
        out_shape=jax.ShapeDtypeStruct((N_GRAPHS, N_EDGES), jnp.float32),
    )(lap_val, alpha)


def _sc_body(pk_hbm, proj_hbm, z_hbm, out_hbm, idx_v, g_v, s_v, proj_sp, acc,
             sg0, sg1, ssA, ssB, si0, si1, si2, si3):
    ss = (ssA, ssB)
    c_ax = lax.axis_index("c")
    s_ax = lax.axis_index("s")
    sg = (sg0, sg1)
    si = (si0, si1, si2, si3)

    rows = pl.ds(s_ax * OUT_ROWS_PER_SUB, OUT_ROWS_PER_SUB)
    pltpu.sync_copy(
        proj_hbm.at[pl.ds(c_ax * N_NODES + s_ax * OUT_ROWS_PER_SUB,
                          OUT_ROWS_PER_SUB)],
        proj_sp.at[rows])
    pltpu.sync_copy(z_hbm.at[rows], acc.at[rows])
    plsc.subcore_barrier()

    base_row = s_ax * N_CHUNKS

    def fire_idx(c, q):
        return pltpu.async_copy(pk_hbm.at[base_row + c], idx_v.at[q], si[q])

    def fire_gather(c, b, q):
        del c
        return pltpu.async_copy(proj_sp.at[idx_v.at[q, 0]], g_v.at[b], sg[b])

    def scale(b, q, h):
        vals_row = idx_v.at[q, 2]

        @plsc.parallel_loop(h * (CHUNK // 2), (h + 1) * (CHUNK // 2),
                            unroll=4)
        def _edge(e):
            vf = plsc.bitcast(
                plsc.load_gather(vals_row, [jnp.full((16,), e, jnp.int32)]),
                jnp.float32)
            grow = g_v.at[b, e]
            srow = s_v.at[h, e - h * (CHUNK // 2)]
            for t in range(4):
                w = plsc.bitcast(grow[pl.ds(t * 16, 16)], jnp.bfloat16)
                a, bb = plsc.unpack(w, format=plsc.PackFormat.INTERLEAVED)
                srow[pl.ds(t * 32, 16)] = a * vf
                srow[pl.ds(t * 32 + 16, 16)] = bb * vf

    def fire_scatter(q, h):
        return pltpu.async_copy(
            s_v.at[h],
            acc.at[idx_v.at[q, 1, pl.ds(h * (CHUNK // 2), CHUNK // 2)]],
            ss[h], add=True)

    cp = fire_idx(0, 0)
    fire_idx(1, 1)
    cp.wait()
    fire_gather(0, 0, 0)

    @pl.loop(0, N_CHUNKS // 4)
    def _block(k):
        c0 = k * 4
        for pp in range(4):
            c = c0 + pp
            b = pp % 2
            q = pp % IRING
            qn = (pp + 1) % IRING
            q2 = (pp + 2) % IRING

            pltpu.make_async_copy(proj_sp.at[idx_v.at[q, 0]], g_v.at[b],
                                  sg[b]).wait()

            @pl.when(c + 1 < N_CHUNKS)
            def _():
                pltpu.make_async_copy(pk_hbm.at[base_row + c + 1],
                                      idx_v.at[qn], si[qn]).wait()
                fire_gather(c + 1, 1 - b, qn)

            for h in range(2):
                @pl.when(c >= 1)
                def _(h=h):
                    pltpu.make_async_copy(
                        s_v.at[h],
                        acc.at[idx_v.at[q2, 1, pl.ds(h * (CHUNK // 2),
                                                     CHUNK // 2)]],
                        ss[h]).wait()

                scale(b, q, h)
                fire_scatter(q, h)

            @pl.when(c + 2 < N_CHUNKS)
            def _():
                fire_idx(c + 2, q2)

    q_last = (N_CHUNKS - 1) % IRING
    for h in range(2):
        pltpu.make_async_copy(
            s_v.at[h],
            acc.at[idx_v.at[q_last, 1, pl.ds(h * (CHUNK // 2), CHUNK // 2)]],
            ss[h]).wait()

    plsc.subcore_barrier()
    pltpu.sync_copy(
        acc.at[rows],
        out_hbm.at[rows, pl.ds(c_ax * HALF, HALF)])


def _spmm(pk, proj_i32, zeros):
    mesh = plsc.VectorSubcoreMesh(
        core_axis_name="c", subcore_axis_name="s", num_cores=2,
        num_subcores=NSUB)
    run = functools.partial(
        pl.kernel,
        out_type=jax.ShapeDtypeStruct((N_NODES, D), jnp.float32),
        mesh=mesh,
        scratch_types=[
            pltpu.VMEM((IRING, 3, CHUNK), jnp.int32),
            pltpu.VMEM((2, CHUNK, HALF // 2), jnp.int32),
            pltpu.VMEM((2, CHUNK // 2, HALF), jnp.float32),
            pltpu.VMEM_SHARED((N_NODES, HALF // 2), jnp.int32),
            pltpu.VMEM_SHARED((N_NODES, HALF), jnp.float32),
        ] + [pltpu.SemaphoreType.DMA] * 8,
        compiler_params=pltpu.CompilerParams(
            use_tc_tiling_on_sc=False, needs_layout_passes=False),
    )(_sc_body)
    return run(pk, proj_i32, zeros)


def kernel(company_emb, lap_idx, lap_val, W, alpha):
    row_order = jnp.asarray(
        np.concatenate([_PERM, HALF + _PERM]), dtype=jnp.int32)
    W2 = W[row_order]
    proj_cat = _project(company_emb, W2)
    proj_i32 = lax.bitcast_convert_type(
        proj_cat.reshape(2 * N_NODES, HALF // 2, 2), jnp.int32)
    vals2 = _scale_vals(lap_val, alpha)

    idx32 = lap_idx.astype(jnp.int32)
    dsts = idx32[:, 0, :].reshape(-1)
    cols = idx32[:, 1, :].reshape(-1)
    valsf = vals2.reshape(-1)
    pad = E_PAD - E_TOT
    d2 = jnp.pad(dsts, (0, pad)).reshape(PK_ROWS, CHUNK)
    c2 = jnp.pad(cols, (0, pad)).reshape(PK_ROWS, CHUNK)
    v2 = lax.bitcast_convert_type(
        jnp.pad(valsf, (0, pad)), jnp.int32).reshape(PK_ROWS, CHUNK)
    pk = jnp.stack([c2, d2, v2], axis=1)
    zeros = jnp.zeros((N_NODES, HALF), jnp.float32)
    return _spmm(pk, proj_i32, zeros)


# speedup vs baseline: 1.1326x; 1.1326x over previous
import functools

import jax
import jax.numpy as jnp
import numpy as np
from jax import lax
from jax.experimental import pallas as pl
from jax.experimental.pallas import tpu as pltpu
from jax.experimental.pallas import tpu_sc as plsc

N_NODES = 10000
D = 256
HALF = 128
N_GRAPHS = 5
N_EDGES = 160000
E_TOT = N_GRAPHS * N_EDGES
NSUB = 16
CHUNK = 40
N_CHUNKS = 1256
E_PAD = NSUB * N_CHUNKS * CHUNK
PK_ROWS = E_PAD // CHUNK
OUT_ROWS_PER_SUB = N_NODES // NSUB
IRING = 4

_PERM = np.arange(HALF)
_PERM = (_PERM // 32) * 32 + (_PERM % 32) // 2 + 16 * (_PERM % 2)


def _proj_body(x_ref, w_ref, o_ref):
    o_ref[...] = lax.dot_general(
        x_ref[...], w_ref[...], (((1,), (1,)), ((), ())),
        precision=lax.Precision.HIGHEST).astype(jnp.bfloat16)


def _project(x, W2):
    RB = 1000
    nblk = N_NODES // RB
    return pl.pallas_call(
        _proj_body,
        grid=(2, nblk),
        in_specs=[
            pl.BlockSpec((RB, D), lambda h, i: (i, 0)),
            pl.BlockSpec((HALF, D), lambda h, i: (h, 0)),
        ],
        out_specs=pl.BlockSpec((RB, HALF), lambda h, i: (h * nblk + i, 0)),
        out_shape=jax.ShapeDtypeStruct((2 * N_NODES, HALF), jnp.bfloat16),
    )(x, W2)


def _vals_body(v_ref, a_ref, o_ref):
    o_ref[...] = v_ref[...] * jax.nn.sigmoid(a_ref[...])


def _scale_vals(lap_val, alpha):
    return pl.pallas_call(
        _vals_body,
        in_specs=[
            pl.BlockSpec((N_GRAPHS, N_EDGES), lambda: (0, 0)),
            pl.BlockSpec((N_GRAPHS, 1), lambda: (0, 0)),
        ],
        out_specs=pl.BlockSpec((N_GRAPHS, N_EDGES), lambda: (0, 0)),
        out_shape=jax.ShapeDtypeStruct((N_GRAPHS, N_EDGES), jnp.float32),
    )(lap_val, alpha)


def _sc_body(pk_hbm, proj_hbm, z_hbm, out_hbm, idx_v, g_v, s_v, proj_sp, acc,
             sg0, sg1, ss0, si0, si1, si2, si3):
    c_ax = lax.axis_index("c")
    s_ax = lax.axis_index("s")
    sg = (sg0, sg1)
    si = (si0, si1, si2, si3)

    rows = pl.ds(s_ax * OUT_ROWS_PER_SUB, OUT_ROWS_PER_SUB)
    pltpu.sync_copy(
        proj_hbm.at[pl.ds(c_ax * N_NODES + s_ax * OUT_ROWS_PER_SUB,
                          OUT_ROWS_PER_SUB)],
        proj_sp.at[rows])
    pltpu.sync_copy(z_hbm.at[rows], acc.at[rows])
    plsc.subcore_barrier()

    base_row = s_ax * N_CHUNKS

    def fire_idx(c, q):
        return pltpu.async_copy(pk_hbm.at[base_row + c], idx_v.at[q], si[q])

    def fire_gather(c, b, q):
        del c
        return pltpu.async_copy(proj_sp.at[idx_v.at[q, 0]], g_v.at[b], sg[b])

    def scale(b, q):
        vals_row = idx_v.at[q, 2]

        @plsc.parallel_loop(0, CHUNK, unroll=8)
        def _edge(e):
            vf = plsc.bitcast(
                plsc.load_gather(vals_row, [jnp.full((16,), e, jnp.int32)]),
                jnp.float32)
            grow = g_v.at[b, e]
            srow = s_v.at[e]
            for t in range(4):
                w = plsc.bitcast(grow[pl.ds(t * 16, 16)], jnp.bfloat16)
                a, bb = plsc.unpack(w, format=plsc.PackFormat.INTERLEAVED)
                srow[pl.ds(t * 32, 16)] = a * vf
                srow[pl.ds(t * 32 + 16, 16)] = bb * vf

    def fire_scatter(q):
        return pltpu.async_copy(s_v, acc.at[idx_v.at[q, 1]], ss0, add=True)

    cp = fire_idx(0, 0)
    fire_idx(1, 1)
    cp.wait()
    fire_gather(0, 0, 0)

    @pl.loop(0, N_CHUNKS // 4)
    def _block(k):
        c0 = k * 4
        for pp in range(4):
            c = c0 + pp
            b = pp % 2
            q = pp % IRING
            qn = (pp + 1) % IRING
            q2 = (pp + 2) % IRING

            pltpu.make_async_copy(proj_sp.at[idx_v.at[q, 0]], g_v.at[b],
                                  sg[b]).wait()

            @pl.when(c + 1 < N_CHUNKS)
            def _():
                pltpu.make_async_copy(pk_hbm.at[base_row + c + 1],
                                      idx_v.at[qn], si[qn]).wait()
                fire_gather(c + 1, 1 - b, qn)

            @pl.when(c >= 1)
            def _():
                pltpu.make_async_copy(s_v, acc.at[idx_v.at[q2, 1]],
                                      ss0).wait()

            scale(b, q)
            fire_scatter(q)

            @pl.when(c + 2 < N_CHUNKS)
            def _():
                fire_idx(c + 2, q2)

    q_last = (N_CHUNKS - 1) % IRING
    pltpu.make_async_copy(s_v, acc.at[idx_v.at[q_last, 1]], ss0).wait()

    plsc.subcore_barrier()
    pltpu.sync_copy(
        acc.at[rows],
        out_hbm.at[rows, pl.ds(c_ax * HALF, HALF)])


def _spmm(pk, proj_i32, zeros):
    mesh = plsc.VectorSubcoreMesh(
        core_axis_name="c", subcore_axis_name="s", num_cores=2,
        num_subcores=NSUB)
    run = functools.partial(
        pl.kernel,
        out_type=jax.ShapeDtypeStruct((N_NODES, D), jnp.float32),
        mesh=mesh,
        scratch_types=[
            pltpu.VMEM((IRING, 3, CHUNK), jnp.int32),
            pltpu.VMEM((2, CHUNK, HALF // 2), jnp.int32),
            pltpu.VMEM((CHUNK, HALF), jnp.float32),
            pltpu.VMEM_SHARED((N_NODES, HALF // 2), jnp.int32),
            pltpu.VMEM_SHARED((N_NODES, HALF), jnp.float32),
        ] + [pltpu.SemaphoreType.DMA] * 7,
        compiler_params=pltpu.CompilerParams(
            use_tc_tiling_on_sc=False, needs_layout_passes=False),
    )(_sc_body)
    return run(pk, proj_i32, zeros)


def kernel(company_emb, lap_idx, lap_val, W, alpha):
    row_order = jnp.asarray(
        np.concatenate([_PERM, HALF + _PERM]), dtype=jnp.int32)
    W2 = W[row_order]
    proj_cat = _project(company_emb, W2)
    proj_i32 = lax.bitcast_convert_type(
        proj_cat.reshape(2 * N_NODES, HALF // 2, 2), jnp.int32)
    vals2 = _scale_vals(lap_val, alpha)

    idx32 = lap_idx.astype(jnp.int32)
    dsts = idx32[:, 0, :].reshape(-1)
    cols = idx32[:, 1, :].reshape(-1)
    valsf = vals2.reshape(-1)
    pad = E_PAD - E_TOT
    d2 = jnp.pad(dsts, (0, pad)).reshape(PK_ROWS, CHUNK)
    c2 = jnp.pad(cols, (0, pad)).reshape(PK_ROWS, CHUNK)
    v2 = lax.bitcast_convert_type(
        jnp.pad(valsf, (0, pad)), jnp.int32).reshape(PK_ROWS, CHUNK)
    pk = jnp.stack([c2, d2, v2], axis=1)
    zeros = jnp.zeros((N_NODES, HALF), jnp.float32)
    return _spmm(pk, proj_i32, zeros)


# speedup vs baseline: 1.2180x; 1.0754x over previous
import functools

import jax
import jax.numpy as jnp
import numpy as np
from jax import lax
from jax.experimental import pallas as pl
from jax.experimental.pallas import tpu as pltpu
from jax.experimental.pallas import tpu_sc as plsc

N_NODES = 10000
D = 256
HALF = 128
N_GRAPHS = 5
N_EDGES = 160000
E_TOT = N_GRAPHS * N_EDGES
NSUB = 16
CHUNK = 40
N_CHUNKS = 1256
E_PAD = NSUB * N_CHUNKS * CHUNK
PK_ROWS = E_PAD // CHUNK
OUT_ROWS_PER_SUB = N_NODES // NSUB
IRING = 4

_PERM = np.arange(HALF)
_PERM = (_PERM // 32) * 32 + (_PERM % 32) // 2 + 16 * (_PERM % 2)


def _proj_body(x_ref, w_ref, o_ref):
    o_ref[...] = lax.dot_general(
        x_ref[...], w_ref[...], (((1,), (1,)), ((), ())),
        precision=lax.Precision.HIGHEST).astype(jnp.bfloat16)


def _project(x, W2):
    RB = 1000
    nblk = N_NODES // RB
    return pl.pallas_call(
        _proj_body,
        grid=(2, nblk),
        in_specs=[
            pl.BlockSpec((RB, D), lambda h, i: (i, 0)),
            pl.BlockSpec((HALF, D), lambda h, i: (h, 0)),
        ],
        out_specs=pl.BlockSpec((RB, HALF), lambda h, i: (h * nblk + i, 0)),
        out_shape=jax.ShapeDtypeStruct((2 * N_NODES, HALF), jnp.bfloat16),
    )(x, W2)


def _vals_body(v_ref, a_ref, o_ref):
    o_ref[...] = v_ref[...] * jax.nn.sigmoid(a_ref[...])


def _scale_vals(lap_val, alpha):
    return pl.pallas_call(
        _vals_body,
        in_specs=[
            pl.BlockSpec((N_GRAPHS, N_EDGES), lambda: (0, 0)),
            pl.BlockSpec((N_GRAPHS, 1), lambda: (0, 0)),
        ],
        out_specs=pl.BlockSpec((N_GRAPHS, N_EDGES), lambda: (0, 0)),
        out_shape=jax.ShapeDtypeStruct((N_GRAPHS, N_EDGES), jnp.float32),
    )(lap_val, alpha)


def _sc_body(pk_hbm, proj_hbm, z_hbm, out_hbm, idx_v, g_v, s_v, proj_sp, acc,
             sg0, sg1, ss0, si0, si1, si2, si3):
    c_ax = lax.axis_index("c")
    s_ax = lax.axis_index("s")
    sg = (sg0, sg1)
    si = (si0, si1, si2, si3)

    rows = pl.ds(s_ax * OUT_ROWS_PER_SUB, OUT_ROWS_PER_SUB)
    pltpu.sync_copy(
        proj_hbm.at[pl.ds(c_ax * N_NODES + s_ax * OUT_ROWS_PER_SUB,
                          OUT_ROWS_PER_SUB)],
        proj_sp.at[rows])
    pltpu.sync_copy(z_hbm.at[rows], acc.at[rows])
    plsc.subcore_barrier()

    base_row = s_ax * N_CHUNKS

    def fire_idx(c, q):
        return pltpu.async_copy(pk_hbm.at[base_row + c], idx_v.at[q], si[q])

    def fire_gather(c, b, q):
        del c
        return pltpu.async_copy(proj_sp.at[idx_v.at[q, 0]], g_v.at[b], sg[b])

    def scale(b, q):
        vals_row = idx_v.at[q, 2]

        @plsc.parallel_loop(0, CHUNK, unroll=8)
        def _edge(e):
            vf = plsc.bitcast(
                plsc.load_gather(vals_row, [jnp.full((16,), e, jnp.int32)]),
                jnp.float32)
            grow = g_v.at[b, e]
            srow = s_v.at[e]
            for t in range(4):
                w = grow[pl.ds(t * 32, 32)]
                a, bb = plsc.unpack(w, format=plsc.PackFormat.INTERLEAVED)
                srow[pl.ds(t * 32, 16)] = a * vf
                srow[pl.ds(t * 32 + 16, 16)] = bb * vf

    def fire_scatter(q):
        return pltpu.async_copy(s_v, acc.at[idx_v.at[q, 1]], ss0, add=True)

    cp = fire_idx(0, 0)
    fire_idx(1, 1)
    cp.wait()
    fire_gather(0, 0, 0)

    @pl.loop(0, N_CHUNKS // 4)
    def _block(k):
        c0 = k * 4
        for pp in range(4):
            c = c0 + pp
            b = pp % 2
            q = pp % IRING
            qn = (pp + 1) % IRING
            q2 = (pp + 2) % IRING

            pltpu.make_async_copy(proj_sp.at[idx_v.at[q, 0]], g_v.at[b],
                                  sg[b]).wait()

            @pl.when(c + 1 < N_CHUNKS)
            def _():
                pltpu.make_async_copy(pk_hbm.at[base_row + c + 1],
                                      idx_v.at[qn], si[qn]).wait()
                fire_gather(c + 1, 1 - b, qn)

            @pl.when(c >= 1)
            def _():
                pltpu.make_async_copy(s_v, acc.at[idx_v.at[q2, 1]],
                                      ss0).wait()

            scale(b, q)
            fire_scatter(q)

            @pl.when(c + 2 < N_CHUNKS)
            def _():
                fire_idx(c + 2, q2)

    q_last = (N_CHUNKS - 1) % IRING
    pltpu.make_async_copy(s_v, acc.at[idx_v.at[q_last, 1]], ss0).wait()

    plsc.subcore_barrier()
    pltpu.sync_copy(
        acc.at[rows],
        out_hbm.at[rows, pl.ds(c_ax * HALF, HALF)])


def _spmm(pk, proj_bf16, zeros):
    mesh = plsc.VectorSubcoreMesh(
        core_axis_name="c", subcore_axis_name="s", num_cores=2,
        num_subcores=NSUB)
    run = functools.partial(
        pl.kernel,
        out_type=jax.ShapeDtypeStruct((N_NODES, D), jnp.float32),
        mesh=mesh,
        scratch_types=[
            pltpu.VMEM((IRING, 3, CHUNK), jnp.int32),
            pltpu.VMEM((2, CHUNK, HALF), jnp.bfloat16),
            pltpu.VMEM((CHUNK, HALF), jnp.float32),
            pltpu.VMEM_SHARED((N_NODES, HALF), jnp.bfloat16),
            pltpu.VMEM_SHARED((N_NODES, HALF), jnp.float32),
        ] + [pltpu.SemaphoreType.DMA] * 7,
        compiler_params=pltpu.CompilerParams(
            use_tc_tiling_on_sc=False, needs_layout_passes=False),
    )(_sc_body)
    return run(pk, proj_bf16, zeros)


def kernel(company_emb, lap_idx, lap_val, W, alpha):
    row_order = jnp.asarray(
        np.concatenate([_PERM, HALF + _PERM]), dtype=jnp.int32)
    W2 = W[row_order]
    proj_cat = _project(company_emb, W2)
    vals2 = _scale_vals(lap_val, alpha)

    idx32 = lap_idx.astype(jnp.int32)
    dsts = idx32[:, 0, :].reshape(-1)
    cols = idx32[:, 1, :].reshape(-1)
    valsf = vals2.reshape(-1)
    pad = E_PAD - E_TOT
    d2 = jnp.pad(dsts, (0, pad)).reshape(PK_ROWS, CHUNK)
    c2 = jnp.pad(cols, (0, pad)).reshape(PK_ROWS, CHUNK)
    v2 = lax.bitcast_convert_type(
        jnp.pad(valsf, (0, pad)), jnp.int32).reshape(PK_ROWS, CHUNK)
    pk = jnp.stack([c2, d2, v2], axis=1)
    zeros = jnp.zeros((N_NODES, HALF), jnp.float32)
    return _spmm(pk, proj_cat, zeros)
